# baseline (speedup 1.0000x reference)
import functools

import jax
import jax.numpy as jnp
from jax import lax
from jax.experimental import pallas as pl
from jax.experimental.pallas import tpu as pltpu
from jax.experimental.pallas import tpu_sc as plsc

VOCAB = 1000000
EMBED = 64
HIDDEN = 64
B = 4096
T = 50
G3 = 3 * HIDDEN
PW = 128

NC = 2
NS = 16
NW = NC * NS
CH = 400

RB = 16384


def _format_body(et_ref, out_ref):
    out_ref[:, 0:EMBED] = et_ref[...].T
    out_ref[:, EMBED:PW] = jnp.zeros((RB, PW - EMBED), jnp.float32)


def _format_table(embT):
    n_steps = (VOCAB + RB - 1) // RB
    return pl.pallas_call(
        _format_body,
        grid=(n_steps,),
        in_specs=[pl.BlockSpec((EMBED, RB), lambda i: (0, i))],
        out_specs=pl.BlockSpec((RB, PW), lambda i: (i, 0)),
        out_shape=jax.ShapeDtypeStruct((VOCAB, PW), jnp.float32),
    )(embT)


def _gather_sc(table, idx_flat):
    TB = idx_flat.shape[0]
    b_per_w = TB // NW
    n_ch = b_per_w // CH
    mesh = plsc.VectorSubcoreMesh(core_axis_name="c", subcore_axis_name="s")

    @functools.partial(
        pl.kernel,
        out_type=jax.ShapeDtypeStruct((TB, PW), jnp.float32),
        mesh=mesh,
        scratch_types=[
            pltpu.VMEM((CH,), jnp.int32),
            pltpu.VMEM((CH, PW), jnp.float32),
            pltpu.SemaphoreType.DMA,
        ],
    )
    def gather_kernel(table_hbm, i_hbm, o_hbm, idx_v, rows_v, sem):
        wid = lax.axis_index("s") * NC + lax.axis_index("c")

        @pl.loop(0, n_ch)
        def _(c):
            base = wid * b_per_w + c * CH
            pltpu.sync_copy(i_hbm.at[pl.ds(base, CH)], idx_v)
            pltpu.async_copy(table_hbm.at[idx_v], rows_v, sem).wait()
            pltpu.sync_copy(rows_v, o_hbm.at[pl.ds(base, CH)])

    return gather_kernel(table, idx_flat)


def _gru_body(xs_hbm, wih_ref, whh_ref, bih_ref, bhh_ref, fcw_ref, fcb_ref,
              out_ref, x0, x1, h_ref, sem0, sem1):
    pltpu.make_async_copy(xs_hbm.at[0], x0, sem0).start()
    pltpu.make_async_copy(xs_hbm.at[1], x1, sem1).start()
    h_ref[...] = jnp.zeros((B, HIDDEN), jnp.float32)
    wih = wih_ref[...]
    whh = whh_ref[...]
    bih = bih_ref[...]
    bhh = bhh_ref[...]

    def gru_step(xt, h):
        gi = jnp.dot(xt, wih, preferred_element_type=jnp.float32) + bih
        gh = jnp.dot(h, whh, preferred_element_type=jnp.float32) + bhh
        r = jax.nn.sigmoid(gi[:, 0:HIDDEN] + gh[:, 0:HIDDEN])
        z = jax.nn.sigmoid(gi[:, HIDDEN:2 * HIDDEN] + gh[:, HIDDEN:2 * HIDDEN])
        n = jnp.tanh(gi[:, 2 * HIDDEN:] + r * gh[:, 2 * HIDDEN:])
        return (1.0 - z) * n + z * h

    def pair(i, carry):
        t0 = 2 * i
        pltpu.make_async_copy(xs_hbm.at[t0], x0, sem0).wait()
        h_ref[...] = gru_step(x0[...], h_ref[...])

        @pl.when(i < (T // 2) - 1)
        def _():
            pltpu.make_async_copy(xs_hbm.at[t0 + 2], x0, sem0).start()

        pltpu.make_async_copy(xs_hbm.at[t0 + 1], x1, sem1).wait()
        h_ref[...] = gru_step(x1[...], h_ref[...])

        @pl.when(i < (T // 2) - 1)
        def _():
            pltpu.make_async_copy(xs_hbm.at[t0 + 3], x1, sem1).start()

        return carry

    lax.fori_loop(0, T // 2, pair, 0)
    out_ref[...] = (
        jnp.dot(h_ref[...], fcw_ref[...], preferred_element_type=jnp.float32)
        + fcb_ref[...]
    )


def _gru_tc(xs, wihT, whhT, bih, bhh, fcwT, fcb):
    return pl.pallas_call(
        _gru_body,
        in_specs=[
            pl.BlockSpec(memory_space=pl.ANY),
            pl.BlockSpec(memory_space=pltpu.MemorySpace.VMEM),
            pl.BlockSpec(memory_space=pltpu.MemorySpace.VMEM),
            pl.BlockSpec(memory_space=pltpu.MemorySpace.VMEM),
            pl.BlockSpec(memory_space=pltpu.MemorySpace.VMEM),
            pl.BlockSpec(memory_space=pltpu.MemorySpace.VMEM),
            pl.BlockSpec(memory_space=pltpu.MemorySpace.VMEM),
        ],
        out_specs=pl.BlockSpec(memory_space=pltpu.MemorySpace.VMEM),
        out_shape=jax.ShapeDtypeStruct((B, HIDDEN), jnp.float32),
        scratch_shapes=[
            pltpu.VMEM((B, PW), jnp.float32),
            pltpu.VMEM((B, PW), jnp.float32),
            pltpu.VMEM((B, HIDDEN), jnp.float32),
            pltpu.SemaphoreType.DMA,
            pltpu.SemaphoreType.DMA,
        ],
    )(xs, wihT, whhT, bih, bhh, fcwT, fcb)


def kernel(x, emb, w_ih, w_hh, b_ih, b_hh, fc_w, fc_b):
    table = _format_table(emb.T)
    idx = x.astype(jnp.int32).T.reshape(-1)
    rows = _gather_sc(table, idx)
    xs = rows.reshape(T, B, PW)
    wih_pad = jnp.concatenate(
        [w_ih.T, jnp.zeros((PW - EMBED, G3), w_ih.dtype)], axis=0)
    return _gru_tc(
        xs,
        wih_pad,
        w_hh.T,
        b_ih.reshape(1, G3),
        b_hh.reshape(1, G3),
        fc_w.T,
        fc_b.reshape(1, HIDDEN),
    )
